# baseline (speedup 1.0000x reference)
import functools

import jax
import jax.numpy as jnp
from jax import lax
from jax.experimental import pallas as pl
from jax.experimental.pallas import tpu as pltpu
from jax.experimental.pallas import tpu_sc as plsc

EMB = 64
ROWBLK = 8
NBUF = 2
NC = 2
NS = 16
NW = NC * NS

TBLK = 512


def _relayout_block(wt_ref, out_ref):
    t = wt_ref[...]
    ident = jnp.eye(EMB, dtype=jnp.float32)
    tt = jax.lax.dot_general(
        t, ident, (((0,), (0,)), ((), ())),
        precision=jax.lax.Precision.HIGHEST,
        preferred_element_type=jnp.float32)
    pair = tt.reshape(TBLK // 2, 2, EMB)
    out_ref[...] = jnp.concatenate([pair[:, 0, :], pair[:, 1, :]], axis=1)


@functools.lru_cache(maxsize=None)
def _make_relayout(vocab: int):
    grid = (vocab + TBLK - 1) // TBLK
    return pl.pallas_call(
        _relayout_block,
        grid=(grid,),
        in_specs=[pl.BlockSpec((EMB, TBLK), lambda i: (0, i))],
        out_specs=pl.BlockSpec((TBLK // 2, 2 * EMB), lambda i: (i, 0)),
        out_shape=jax.ShapeDtypeStruct((vocab // 2, 2 * EMB), jnp.float32),
    )


@functools.lru_cache(maxsize=None)
def _make_gather(batch: int, seq: int, vocab: int):
    rows_w = batch // NW
    nsteps = rows_w // ROWBLK
    assert rows_w * NW == batch and nsteps * ROWBLK == rows_w
    nq = EMB // 16

    mesh = plsc.VectorSubcoreMesh(core_axis_name="c", subcore_axis_name="s")

    @functools.partial(
        pl.kernel,
        mesh=mesh,
        out_type=jax.ShapeDtypeStruct((batch, seq, EMB), jnp.float32),
        compiler_params=pltpu.CompilerParams(use_tc_tiling_on_sc=False),
        scratch_types=[
            pltpu.VMEM((rows_w, seq), jnp.int32),
            pltpu.VMEM((ROWBLK, seq, EMB), jnp.float32),
            pltpu.VMEM((ROWBLK, seq, EMB), jnp.float32),
            pltpu.VMEM((EMB,), jnp.float32),
            pltpu.SemaphoreType.DMA,
            pltpu.SemaphoreType.DMA,
        ],
    )
    def body(x_hbm, wemb_hbm, wpos_hbm, out_hbm, idx_v, buf0, buf1, wpos_v,
             sem0, sem1):
        wid = lax.axis_index("s") * NC + lax.axis_index("c")
        row0 = wid * rows_w
        pltpu.sync_copy(x_hbm.at[pl.ds(row0, rows_w)], idx_v)
        pltpu.sync_copy(wpos_hbm, wpos_v)
        wp = [wpos_v[pl.ds(16 * q, 16)] for q in range(nq)]
        bufs = (buf0, buf1)
        sems = (sem0, sem1)

        def issue(i, b):
            for a in range(ROWBLK):
                pltpu.async_copy(
                    wemb_hbm.at[idx_v.at[i * ROWBLK + a]],
                    bufs[b].at[a], sems[b])

        def wait(b):
            for a in range(ROWBLK):
                pltpu.make_async_copy(
                    wemb_hbm.at[idx_v.at[0]], bufs[b].at[a], sems[b]).wait()

        def process(i, b):
            buf = bufs[b]

            def addrow(r, carry):
                for a in range(ROWBLK):
                    for q in range(nq):
                        sl = pl.ds(16 * q, 16)
                        buf[a, r, sl] = buf[a, r, sl] + wp[q]
                return carry

            lax.fori_loop(0, seq, addrow, 0)
            pltpu.sync_copy(
                buf, out_hbm.at[pl.ds(row0 + i * ROWBLK, ROWBLK)])

        for b in range(NBUF):
            issue(b, b)

        @pl.loop(0, (nsteps - NBUF) // NBUF)
        def main(k):
            i0 = k * NBUF
            for b in range(NBUF):
                i = i0 + b
                wait(b)
                process(i, b)
                issue(i + NBUF, b)

        for b in range(NBUF):
            i = nsteps - NBUF + b
            wait(b)
            process(i, b)

    return body


def kernel(x, W_emb, W_pos):
    b, l = x.shape
    vocab, emb = W_emb.shape
    w_flat = _make_relayout(vocab)(jnp.transpose(W_emb))
    w_rows = w_flat.reshape(vocab, emb)
    return _make_gather(b, l, vocab)(x.astype(jnp.int32), w_rows, W_pos)


# speedup vs baseline: 3.8167x; 3.8167x over previous
import functools

import jax
import jax.numpy as jnp
from jax import lax
from jax.experimental import pallas as pl
from jax.experimental.pallas import tpu as pltpu
from jax.experimental.pallas import tpu_sc as plsc

EMB = 64
ROWBLK = 8
NBUF = 2
NC = 2
NS = 16
NW = NC * NS

TBLK = 8192
HALF = TBLK // 2


def _relayout_block(wt_ref, out_ref):
    t = wt_ref[...]
    tt = jnp.transpose(t)
    out_ref[:, 0:EMB] = tt[:HALF, :]
    out_ref[:, EMB:2 * EMB] = tt[HALF:, :]


@functools.lru_cache(maxsize=None)
def _make_relayout(vocab: int):
    grid = (vocab + TBLK - 1) // TBLK
    return pl.pallas_call(
        _relayout_block,
        grid=(grid,),
        in_specs=[pl.BlockSpec((EMB, TBLK), lambda i: (0, i))],
        out_specs=pl.BlockSpec((HALF, 2 * EMB), lambda i: (i, 0)),
        out_shape=jax.ShapeDtypeStruct((grid * HALF, 2 * EMB), jnp.float32),
    )


@functools.lru_cache(maxsize=None)
def _make_gather(batch: int, seq: int, vocab2: int):
    rows_w = batch // NW
    nsteps = rows_w // ROWBLK
    assert rows_w * NW == batch and nsteps * ROWBLK == rows_w
    nq = EMB // 16
    nchunk = (rows_w * seq) // 16

    mesh = plsc.VectorSubcoreMesh(core_axis_name="c", subcore_axis_name="s")

    @functools.partial(
        pl.kernel,
        mesh=mesh,
        out_type=jax.ShapeDtypeStruct((batch, seq, EMB), jnp.float32),
        compiler_params=pltpu.CompilerParams(
            use_tc_tiling_on_sc=False, needs_layout_passes=False),
        scratch_types=[
            pltpu.VMEM((rows_w, seq), jnp.int32),
            pltpu.VMEM((rows_w, seq), jnp.int32),
            pltpu.VMEM((ROWBLK, seq, EMB), jnp.float32),
            pltpu.VMEM((ROWBLK, seq, EMB), jnp.float32),
            pltpu.VMEM((EMB,), jnp.float32),
            pltpu.SemaphoreType.DMA,
            pltpu.SemaphoreType.DMA,
        ],
    )
    def body(x_hbm, wemb_hbm, wpos_hbm, out_hbm, idx_v, r_v, buf0, buf1,
             wpos_v, sem0, sem1):
        wid = lax.axis_index("s") * NC + lax.axis_index("c")
        row0 = wid * rows_w
        pltpu.sync_copy(x_hbm.at[pl.ds(row0, rows_w)], idx_v)
        pltpu.sync_copy(wpos_hbm, wpos_v)
        wp = [wpos_v[pl.ds(16 * q, 16)] for q in range(nq)]
        bufs = (buf0, buf1)
        sems = (sem0, sem1)

        def remap(tchunk, carry):
            tv = tchunk * 16 + lax.iota(jnp.int32, 16)
            a = lax.shift_right_logical(tv * 5243, 18)
            r = tv - a * seq
            i = plsc.load_gather(idx_v, [a, r])
            blk = lax.shift_left(lax.shift_right_logical(i, 13), 13)
            off = lax.shift_left(jnp.bitwise_and(i, 4095), 1)
            h = jnp.bitwise_and(lax.shift_right_logical(i, 12), 1)
            plsc.store_scatter(r_v, [a, r], blk | off | h)
            return carry

        lax.fori_loop(0, nchunk, remap, 0)

        def issue(i, b):
            for a in range(ROWBLK):
                pltpu.async_copy(
                    wemb_hbm.at[r_v.at[i * ROWBLK + a]],
                    bufs[b].at[a], sems[b])

        def wait(b):
            for a in range(ROWBLK):
                pltpu.make_async_copy(
                    wemb_hbm.at[r_v.at[0]], bufs[b].at[a], sems[b]).wait()

        def process(i, b):
            buf = bufs[b]

            def addrow(r, carry):
                for a in range(ROWBLK):
                    for q in range(nq):
                        sl = pl.ds(16 * q, 16)
                        buf[a, r, sl] = buf[a, r, sl] + wp[q]
                return carry

            lax.fori_loop(0, seq, addrow, 0)
            pltpu.sync_copy(
                buf, out_hbm.at[pl.ds(row0 + i * ROWBLK, ROWBLK)])

        for b in range(NBUF):
            issue(b, b)

        @pl.loop(0, (nsteps - NBUF) // NBUF)
        def main(k):
            i0 = k * NBUF
            for b in range(NBUF):
                i = i0 + b
                wait(b)
                process(i, b)
                issue(i + NBUF, b)

        for b in range(NBUF):
            i = nsteps - NBUF + b
            wait(b)
            process(i, b)

    return body


def kernel(x, W_emb, W_pos):
    b, l = x.shape
    vocab, emb = W_emb.shape
    w_flat = _make_relayout(vocab)(jnp.transpose(W_emb))
    n2 = w_flat.shape[0] * 2
    w_rows = w_flat.reshape(n2, emb)
    return _make_gather(b, l, n2)(x.astype(jnp.int32), w_rows, W_pos)


# speedup vs baseline: 4.3590x; 1.1421x over previous
import functools

import jax
import jax.numpy as jnp
from jax import lax
from jax.experimental import pallas as pl
from jax.experimental.pallas import tpu as pltpu
from jax.experimental.pallas import tpu_sc as plsc

EMB = 64
ROWBLK = 8
NBUF = 2
NC = 2
NS = 16
NW = NC * NS

TBLK = 8192
HALF = TBLK // 2


def _relayout_block(wt_ref, out_ref):
    t = wt_ref[...]
    tt = jnp.transpose(t)
    out_ref[:, 0:EMB] = tt[:HALF, :]
    out_ref[:, EMB:2 * EMB] = tt[HALF:, :]


@functools.lru_cache(maxsize=None)
def _make_relayout(vocab: int):
    grid = (vocab + TBLK - 1) // TBLK
    return pl.pallas_call(
        _relayout_block,
        grid=(grid,),
        in_specs=[pl.BlockSpec((EMB, TBLK), lambda i: (0, i))],
        out_specs=pl.BlockSpec((HALF, 2 * EMB), lambda i: (i, 0)),
        out_shape=jax.ShapeDtypeStruct((grid * HALF, 2 * EMB), jnp.float32),
    )


@functools.lru_cache(maxsize=None)
def _make_gather(batch: int, seq: int, vocab2: int):
    rows_w = batch // NW
    nsteps = rows_w // ROWBLK
    assert rows_w * NW == batch and nsteps * ROWBLK == rows_w
    nq = EMB // 16
    nchunk = (rows_w * seq) // 16

    mesh = plsc.VectorSubcoreMesh(core_axis_name="c", subcore_axis_name="s")

    @functools.partial(
        pl.kernel,
        mesh=mesh,
        out_type=jax.ShapeDtypeStruct((batch, seq, EMB), jnp.float32),
        compiler_params=pltpu.CompilerParams(
            use_tc_tiling_on_sc=False, needs_layout_passes=False),
        scratch_types=[
            pltpu.VMEM((rows_w, seq), jnp.int32),
            pltpu.VMEM((rows_w, seq), jnp.int32),
            pltpu.VMEM((ROWBLK, seq, EMB), jnp.float32),
            pltpu.VMEM((ROWBLK, seq, EMB), jnp.float32),
            pltpu.VMEM((EMB,), jnp.float32),
            pltpu.SemaphoreType.DMA,
            pltpu.SemaphoreType.DMA,
        ],
    )
    def body(x_hbm, wemb_hbm, wpos_hbm, out_hbm, idx_v, r_v, buf0, buf1,
             wpos_v, sem0, sem1):
        wid = lax.axis_index("s") * NC + lax.axis_index("c")
        row0 = wid * rows_w
        pltpu.sync_copy(x_hbm.at[pl.ds(row0, rows_w)], idx_v)
        pltpu.sync_copy(wpos_hbm, wpos_v)
        wp = [wpos_v[pl.ds(16 * q, 16)] for q in range(nq)]
        bufs = (buf0, buf1)
        sems = (sem0, sem1)

        def remap(tchunk, carry):
            tv = tchunk * 16 + lax.iota(jnp.int32, 16)
            a = lax.shift_right_logical(tv * 5243, 18)
            r = tv - a * seq
            i = plsc.load_gather(idx_v, [a, r])
            blk = lax.shift_left(lax.shift_right_logical(i, 13), 13)
            off = lax.shift_left(jnp.bitwise_and(i, 4095), 1)
            h = jnp.bitwise_and(lax.shift_right_logical(i, 12), 1)
            plsc.store_scatter(r_v, [a, r], blk | off | h)
            return carry

        lax.fori_loop(0, nchunk, remap, 0)

        def issue(i, b):
            for a in range(ROWBLK):
                pltpu.async_copy(
                    wemb_hbm.at[r_v.at[i * ROWBLK + a]],
                    bufs[b].at[a], sems[b])

        def wait(b):
            for a in range(ROWBLK):
                pltpu.make_async_copy(
                    wemb_hbm.at[r_v.at[0]], bufs[b].at[a], sems[b]).wait()

        def process(i, b):
            buf = bufs[b]

            def addrow(r, carry):
                for a in range(ROWBLK):
                    for q in range(nq):
                        sl = pl.ds(16 * q, 16)
                        buf[a, r, sl] = buf[a, r, sl] + wp[q]
                return carry

            lax.fori_loop(0, seq, addrow, 0)
            pltpu.sync_copy(
                buf, out_hbm.at[pl.ds(row0 + i * ROWBLK, ROWBLK)])

        for b in range(NBUF):
            issue(b, b)

        @pl.loop(0, (nsteps - NBUF) // NBUF)
        def main(k):
            i0 = k * NBUF
            for b in range(NBUF):
                i = i0 + b
                wait(b)
                process(i, b)
                issue(i + NBUF, b)

        for b in range(NBUF):
            i = nsteps - NBUF + b
            wait(b)
            process(i, b)

    return body


BBLK = 128


def _out_block(in_ref, out_ref):
    t3 = in_ref[...].reshape(BBLK, 25, 128)
    for l in range(50):
        sel = t3[:, l // 2, 64 * (l % 2):64 * (l % 2) + 64]
        out_ref[l, :, :] = jnp.transpose(sel)


@functools.lru_cache(maxsize=None)
def _make_out_relayout(batch: int, seq: int):
    grid = batch // BBLK
    rows = BBLK * seq * EMB // 128
    return pl.pallas_call(
        _out_block,
        grid=(grid,),
        in_specs=[pl.BlockSpec((rows, 128), lambda i: (i, 0))],
        out_specs=pl.BlockSpec((seq, EMB, BBLK), lambda i: (0, 0, i)),
        out_shape=jax.ShapeDtypeStruct((seq, EMB, batch), jnp.float32),
    )


def kernel(x, W_emb, W_pos):
    b, l = x.shape
    vocab, emb = W_emb.shape
    w_flat = _make_relayout(vocab)(jnp.transpose(W_emb))
    n2 = w_flat.shape[0] * 2
    w_rows = w_flat.reshape(n2, emb)
    sc_out = _make_gather(b, l, n2)(x.astype(jnp.int32), w_rows, W_pos)
    flat = sc_out.reshape(b * l * emb // 128, 128)
    out_t = _make_out_relayout(b, l)(flat)
    return jnp.transpose(out_t, (2, 0, 1))
